# baseline (speedup 1.0000x reference)
import jax
import jax.numpy as jnp
from jax import lax
from jax.experimental import pallas as pl
from jax.experimental.pallas import tpu as pltpu
from jax.experimental.pallas import tpu_sc as plsc

B = 4096
N_NUM = 13
N_CAT = 26
SEQ_LEN = 50
VOCAB = 100000
D = 32
NCOLS = N_NUM + N_CAT + SEQ_LEN
NSLOT = N_NUM + N_CAT + 1
SUPW = 128
SPF = VOCAB // 4
TBL_SUP = (N_CAT + 1) * SPF

NC, NS = 2, 16
NW = NC * NS

VB = 512
NFULL = VOCAB // VB
VT = 128
NBLK = (N_CAT + 1) * NFULL
BSUP = VB * D // SUPW
TSUP = VT * D // SUPW
NLAST = (VOCAB - NFULL * VB - VT) // 4


def _transpose_block(inb, outb, nsup, lane):
  def s_body(s, c2):
    for k in range(4):
      v = s * 4 + k
      cols = jnp.full((16,), v, jnp.int32)
      outb[s, pl.ds(k * D, 16)] = plsc.load_gather(inb, [lane, cols])
      outb[s, pl.ds(k * D + 16, 16)] = plsc.load_gather(
          inb, [lane + 16, cols])
    return c2
  lax.fori_loop(0, nsup, s_body, 0, unroll=4)


def _conv_body(cat_hbm, seq_hbm, tail_hbm, tbl_hbm,
               in0, in1, out0, out1, tin2, tout2, tbuf,
               sem_i0, sem_i1, sem_o0, sem_o1):
  wid = lax.axis_index("s") * NC + lax.axis_index("c")
  lane = lax.iota(jnp.int32, 16)
  nblk = (NBLK - 1 - wid) // NW + 1

  ins = (in0, in1)
  outs = (out0, out1)
  sem_is = (sem_i0, sem_i1)
  sem_os = (sem_o0, sem_o1)

  def fire_in(g, buf, sem):
    bid = g * NW + wid
    f = bid // NFULL
    cb = bid - f * NFULL

    @pl.when(f < N_CAT)
    def _():
      pltpu.async_copy(cat_hbm.at[f, pl.ds(0, D), pl.ds(cb * VB, VB)],
                       buf, sem)

    @pl.when(f >= N_CAT)
    def _():
      pltpu.async_copy(seq_hbm.at[pl.ds(0, D), pl.ds(cb * VB, VB)],
                       buf, sem)

  def g_body(g, c2):
    for p in range(2):
      @pl.when(g % 2 == p)
      def _():
        @pl.when(g + 1 < nblk)
        def _():
          fire_in(g + 1, ins[1 - p], sem_is[1 - p])
        pltpu.make_async_copy(
            cat_hbm.at[0, pl.ds(0, D), pl.ds(0, VB)], ins[p],
            sem_is[p]).wait()

        @pl.when(g >= 2)
        def _():
          pltpu.make_async_copy(
              outs[p], tbl_hbm.at[pl.ds(0, BSUP)], sem_os[p]).wait()
        _transpose_block(ins[p], outs[p], BSUP, lane)
        bid = g * NW + wid
        f = bid // NFULL
        cb = bid - f * NFULL
        pltpu.async_copy(
            outs[p], tbl_hbm.at[pl.ds(f * SPF + cb * BSUP, BSUP)],
            sem_os[p])
    return c2

  fire_in(0, ins[0], sem_is[0])
  lax.fori_loop(0, nblk, g_body, 0)

  for p in range(2):
    @pl.when((nblk - 1) % 2 == p)
    def _():
      pltpu.make_async_copy(
          outs[p], tbl_hbm.at[pl.ds(0, BSUP)], sem_os[p]).wait()

    @pl.when((nblk - 2) % 2 == p)
    def _():
      pltpu.make_async_copy(
          outs[p], tbl_hbm.at[pl.ds(0, BSUP)], sem_os[p]).wait()

  @pl.when(wid <= N_CAT)
  def _():
    f = wid

    @pl.when(f < N_CAT)
    def _():
      pltpu.sync_copy(cat_hbm.at[f, pl.ds(0, D), pl.ds(NFULL * VB, VT)], tin2)

    @pl.when(f >= N_CAT)
    def _():
      pltpu.sync_copy(seq_hbm.at[pl.ds(0, D), pl.ds(NFULL * VB, VT)], tin2)
    _transpose_block(tin2, tout2, TSUP, lane)
    pltpu.sync_copy(tout2, tbl_hbm.at[pl.ds(f * SPF + NFULL * BSUP, TSUP)])
    pltpu.sync_copy(tail_hbm.at[pl.ds(f * NLAST, NLAST)], tbuf)
    pltpu.sync_copy(tbuf,
                    tbl_hbm.at[pl.ds(f * SPF + NFULL * BSUP + TSUP, NLAST)])


_convert = pl.kernel(
    _conv_body,
    out_type=jax.ShapeDtypeStruct((TBL_SUP, SUPW), jnp.float32),
    mesh=plsc.VectorSubcoreMesh(core_axis_name="c", subcore_axis_name="s"),
    compiler_params=pltpu.CompilerParams(
        needs_layout_passes=False, use_tc_tiling_on_sc=True),
    scratch_types=[
        pltpu.VMEM((D, VB), jnp.float32),
        pltpu.VMEM((D, VB), jnp.float32),
        pltpu.VMEM((BSUP, SUPW), jnp.float32),
        pltpu.VMEM((BSUP, SUPW), jnp.float32),
        pltpu.VMEM((D, VT), jnp.float32),
        pltpu.VMEM((TSUP, SUPW), jnp.float32),
        pltpu.VMEM((NLAST, SUPW), jnp.float32),
        pltpu.SemaphoreType.DMA,
        pltpu.SemaphoreType.DMA,
        pltpu.SemaphoreType.DMA,
        pltpu.SemaphoreType.DMA,
    ],
)

RPW = B // NW
C = 16
NCHUNK = RPW // C
RPS = NSLOT * D // SUPW

CAT_SUP = N_CAT * C
CAT_H = CAT_SUP // 2
SEQ_SUP = SEQ_LEN * C
SEQ_H = SEQ_SUP // 2
OUT_SUP = C * RPS


def _fire_gather(tbl, idx_ref, base, n, buf, sem):
  descs = []
  off = 0
  while off < n:
    m = min(128, n - off)
    descs.append(pltpu.async_copy(
        tbl.at[idx_ref.at[pl.ds(base + off, m)]],
        buf.at[pl.ds(off, m)], sem))
    off += m
  return descs


def _body(x_hbm, w_hbm, tbl_hbm, out_hbm,
          xv, seq_v, cat_v, out_v, gidx_seq, gidx_cat, rcp_v, wv,
          sem_seq, sem_cat, sem_out):
  wid = lax.axis_index("s") * NC + lax.axis_index("c")
  lane = lax.iota(jnp.int32, 16)

  pltpu.sync_copy(w_hbm, wv)

  def chunk_body(ci, carry):
    gbase = wid * RPW + ci * C

    pltpu.sync_copy(x_hbm.at[pl.ds(gbase * NCOLS, C * NCOLS)], xv)

    rows89 = lane * NCOLS

    for j in range(N_CAT):
      v = plsc.load_gather(xv, [rows89 + (N_NUM + j)])
      gidx_cat[pl.ds(j * C, C)] = j * SPF + lax.shift_right_logical(v, 2)
    cnt = jnp.zeros((16,), jnp.float32)
    for t in range(SEQ_LEN):
      v = plsc.load_gather(xv, [rows89 + (N_NUM + N_CAT + t)])
      gidx_seq[pl.ds(t * C, C)] = (
          N_CAT * SPF + lax.shift_right_logical(v, 2))
      cnt = cnt + jnp.where(v != 0, 1.0, 0.0)
    rcp_v[pl.ds(0, 16)] = 1.0 / jnp.maximum(cnt, 1e-12)

    seq_descs = _fire_gather(tbl_hbm, gidx_seq, 0, SEQ_H, seq_v, sem_seq)
    cat_descs = _fire_gather(tbl_hbm, gidx_cat, 0, CAT_H, cat_v, sem_cat)

    for dsc in seq_descs:
      dsc.wait()

    def nrow_body(b, c2):
      xoff = b * NCOLS
      orow = b * RPS
      for i in range(N_NUM):
        xi = plsc.load_gather(
            xv, [jnp.full((16,), xoff + i, jnp.int32)]).astype(jnp.float32)
        r = orow + (i * D) // SUPW
        c0 = (i * D) % SUPW
        out_v[r, pl.ds(c0, 16)] = xi * wv[pl.ds(i * D, 16)]
        out_v[r, pl.ds(c0 + 16, 16)] = xi * wv[pl.ds(i * D + 16, 16)]
      acc0 = jnp.zeros((16,), jnp.float32)
      acc1 = jnp.zeros((16,), jnp.float32)
      for t in range(SEQ_LEN // 2):
        xq = plsc.load_gather(
            xv, [jnp.full((16,), xoff + N_NUM + N_CAT + t, jnp.int32)])
        cq = lax.bitwise_and(xq, 3) * D + lane
        rr = jnp.full((16,), t * C + b, jnp.int32)
        acc0 = acc0 + plsc.load_gather(seq_v, [rr, cq])
        acc1 = acc1 + plsc.load_gather(seq_v, [rr, cq + 16])
      out_v[orow + RPS - 1, pl.ds(96, 16)] = acc0
      out_v[orow + RPS - 1, pl.ds(112, 16)] = acc1
      return c2

    lax.fori_loop(0, C, nrow_body, 0)

    seq_descs = _fire_gather(tbl_hbm, gidx_seq, SEQ_H, SEQ_H, seq_v, sem_seq)

    for dsc in cat_descs:
      dsc.wait()

    def cat_extract(j0, b, _):
      for j in range(j0, j0 + N_CAT // 2):
        slot = N_NUM + j
        row = (j - j0) * C + b
        xq = plsc.load_gather(
            xv, [jnp.full((16,), b * NCOLS + N_NUM + j, jnp.int32)])
        cq = lax.bitwise_and(xq, 3) * D + lane
        rr = jnp.full((16,), row, jnp.int32)
        r = b * RPS + (slot * D) // SUPW
        c0 = (slot * D) % SUPW
        out_v[r, pl.ds(c0, 16)] = plsc.load_gather(cat_v, [rr, cq])
        out_v[r, pl.ds(c0 + 16, 16)] = plsc.load_gather(cat_v, [rr, cq + 16])
      return _

    lax.fori_loop(0, C, lambda b, c2: cat_extract(0, b, c2), 0)

    cat_descs = _fire_gather(tbl_hbm, gidx_cat, CAT_H, CAT_H, cat_v, sem_cat)

    for dsc in seq_descs:
      dsc.wait()

    def srow_body(b, c2):
      xoff = b * NCOLS
      acc0 = jnp.zeros((16,), jnp.float32)
      acc1 = jnp.zeros((16,), jnp.float32)
      for t in range(SEQ_LEN // 2, SEQ_LEN):
        xq = plsc.load_gather(
            xv, [jnp.full((16,), xoff + N_NUM + N_CAT + t, jnp.int32)])
        cq = lax.bitwise_and(xq, 3) * D + lane
        rr = jnp.full((16,), (t - SEQ_LEN // 2) * C + b, jnp.int32)
        acc0 = acc0 + plsc.load_gather(seq_v, [rr, cq])
        acc1 = acc1 + plsc.load_gather(seq_v, [rr, cq + 16])
      rcp = plsc.load_gather(rcp_v, [jnp.full((16,), b, jnp.int32)])
      r = b * RPS + RPS - 1
      out_v[r, pl.ds(96, 16)] = (out_v[r, pl.ds(96, 16)] + acc0) * rcp
      out_v[r, pl.ds(112, 16)] = (out_v[r, pl.ds(112, 16)] + acc1) * rcp
      return c2

    lax.fori_loop(0, C, srow_body, 0)

    for dsc in cat_descs:
      dsc.wait()
    lax.fori_loop(0, C, lambda b, c2: cat_extract(N_CAT // 2, b, c2), 0)

    pltpu.async_copy(out_v, out_hbm.at[pl.ds(gbase * RPS, OUT_SUP)],
                     sem_out).wait()
    return carry

  lax.fori_loop(0, NCHUNK, chunk_body, 0)


_sc_call = pl.kernel(
    _body,
    out_type=jax.ShapeDtypeStruct((B * RPS, SUPW), jnp.float32),
    mesh=plsc.VectorSubcoreMesh(core_axis_name="c", subcore_axis_name="s"),
    compiler_params=pltpu.CompilerParams(
        needs_layout_passes=False, use_tc_tiling_on_sc=True),
    scratch_types=[
        pltpu.VMEM((C * NCOLS,), jnp.int32),
        pltpu.VMEM((SEQ_H, SUPW), jnp.float32),
        pltpu.VMEM((CAT_H, SUPW), jnp.float32),
        pltpu.VMEM((OUT_SUP, SUPW), jnp.float32),
        pltpu.VMEM((SEQ_SUP,), jnp.int32),
        pltpu.VMEM((CAT_SUP,), jnp.int32),
        pltpu.VMEM((16,), jnp.float32),
        pltpu.VMEM((N_NUM * D,), jnp.float32),
        pltpu.SemaphoreType.DMA,
        pltpu.SemaphoreType.DMA,
        pltpu.SemaphoreType.DMA,
    ],
)


@jax.jit
def kernel(X, W_num, cat_tables, seq_table):
  cat_t = jnp.transpose(cat_tables, (0, 2, 1))
  seq_t = jnp.transpose(seq_table, (1, 0))
  nv = NFULL * VB + VT
  tail = jnp.concatenate([
      cat_tables[:, nv:, :].reshape(N_CAT * NLAST, SUPW),
      seq_table[nv:, :].reshape(NLAST, SUPW)])
  tbl = _convert(cat_t, seq_t, tail)
  out = _sc_call(X.reshape(B * NCOLS), W_num.reshape(N_NUM * D), tbl)
  return out.reshape(B, NSLOT, D)


# speedup vs baseline: 2.4371x; 2.4371x over previous
import jax
import jax.numpy as jnp
from jax import lax
from jax.experimental import pallas as pl
from jax.experimental.pallas import tpu as pltpu
from jax.experimental.pallas import tpu_sc as plsc

B = 4096
N_NUM = 13
N_CAT = 26
SEQ_LEN = 50
VOCAB = 100000
D = 32
NCOLS = N_NUM + N_CAT + SEQ_LEN
NSLOT = N_NUM + N_CAT + 1
SUPW = 128
SPF = VOCAB // 4
TBL_SUP = (N_CAT + 1) * SPF

NC, NS = 2, 16
NW = NC * NS

VB = 512
NFULL = VOCAB // VB
VT = 128
NBLK = (N_CAT + 1) * NFULL
BSUP = VB * D // SUPW
TSUP = VT * D // SUPW
NLAST = (VOCAB - NFULL * VB - VT) // 4


def _transpose_block(inb, outb, ntc, lane, prs, prs32):
  def tc_body(tc, c2):
    v0 = tc * 16
    for dt in range(2):
      rows = lane + dt * 16
      fbase = v0 * D + dt * 16
      for r in range(16):
        x = plsc.load_gather(inb, [rows, prs[r] + v0])
        f = prs32[r] + lane + fbase
        plsc.store_scatter(
            outb, [lax.shift_right_logical(f, 7), lax.bitwise_and(f, 127)], x)
    return c2
  lax.fori_loop(0, ntc, tc_body, 0)


def _conv_body(cat_hbm, seq_hbm, tail_hbm, tbl_hbm,
               in0, in1, out0, out1, tin2, tout2, tbuf,
               sem_i0, sem_i1, sem_o0, sem_o1):
  wid = lax.axis_index("s") * NC + lax.axis_index("c")
  lane = lax.iota(jnp.int32, 16)
  prs = [lax.rem(lane + r, 16) for r in range(16)]
  prs32 = [p32 * D for p32 in prs]
  nblk = (NBLK - 1 - wid) // NW + 1

  ins = (in0, in1)
  outs = (out0, out1)
  sem_is = (sem_i0, sem_i1)
  sem_os = (sem_o0, sem_o1)

  def fire_in(g, buf, sem):
    bid = g * NW + wid
    f = bid // NFULL
    cb = bid - f * NFULL

    @pl.when(f < N_CAT)
    def _():
      pltpu.async_copy(cat_hbm.at[f, pl.ds(0, D), pl.ds(cb * VB, VB)],
                       buf, sem)

    @pl.when(f >= N_CAT)
    def _():
      pltpu.async_copy(seq_hbm.at[pl.ds(0, D), pl.ds(cb * VB, VB)],
                       buf, sem)

  def g_body(g, c2):
    for p in range(2):
      @pl.when(g % 2 == p)
      def _():
        @pl.when(g + 1 < nblk)
        def _():
          fire_in(g + 1, ins[1 - p], sem_is[1 - p])
        pltpu.make_async_copy(
            cat_hbm.at[0, pl.ds(0, D), pl.ds(0, VB)], ins[p],
            sem_is[p]).wait()

        @pl.when(g >= 2)
        def _():
          pltpu.make_async_copy(
              outs[p], tbl_hbm.at[pl.ds(0, BSUP)], sem_os[p]).wait()
        _transpose_block(ins[p], outs[p], VB // 16, lane, prs, prs32)
        bid = g * NW + wid
        f = bid // NFULL
        cb = bid - f * NFULL
        pltpu.async_copy(
            outs[p], tbl_hbm.at[pl.ds(f * SPF + cb * BSUP, BSUP)],
            sem_os[p])
    return c2

  fire_in(0, ins[0], sem_is[0])
  lax.fori_loop(0, nblk, g_body, 0)

  for p in range(2):
    @pl.when((nblk - 1) % 2 == p)
    def _():
      pltpu.make_async_copy(
          outs[p], tbl_hbm.at[pl.ds(0, BSUP)], sem_os[p]).wait()

    @pl.when((nblk - 2) % 2 == p)
    def _():
      pltpu.make_async_copy(
          outs[p], tbl_hbm.at[pl.ds(0, BSUP)], sem_os[p]).wait()

  @pl.when(wid <= N_CAT)
  def _():
    f = wid

    @pl.when(f < N_CAT)
    def _():
      pltpu.sync_copy(cat_hbm.at[f, pl.ds(0, D), pl.ds(NFULL * VB, VT)], tin2)

    @pl.when(f >= N_CAT)
    def _():
      pltpu.sync_copy(seq_hbm.at[pl.ds(0, D), pl.ds(NFULL * VB, VT)], tin2)
    _transpose_block(tin2, tout2, VT // 16, lane, prs, prs32)
    pltpu.sync_copy(tout2, tbl_hbm.at[pl.ds(f * SPF + NFULL * BSUP, TSUP)])
    pltpu.sync_copy(tail_hbm.at[pl.ds(f * NLAST, NLAST)], tbuf)
    pltpu.sync_copy(tbuf,
                    tbl_hbm.at[pl.ds(f * SPF + NFULL * BSUP + TSUP, NLAST)])


_convert = pl.kernel(
    _conv_body,
    out_type=jax.ShapeDtypeStruct((TBL_SUP, SUPW), jnp.float32),
    mesh=plsc.VectorSubcoreMesh(core_axis_name="c", subcore_axis_name="s"),
    compiler_params=pltpu.CompilerParams(
        needs_layout_passes=False, use_tc_tiling_on_sc=True),
    scratch_types=[
        pltpu.VMEM((D, VB), jnp.float32),
        pltpu.VMEM((D, VB), jnp.float32),
        pltpu.VMEM((BSUP, SUPW), jnp.float32),
        pltpu.VMEM((BSUP, SUPW), jnp.float32),
        pltpu.VMEM((D, VT), jnp.float32),
        pltpu.VMEM((TSUP, SUPW), jnp.float32),
        pltpu.VMEM((NLAST, SUPW), jnp.float32),
        pltpu.SemaphoreType.DMA,
        pltpu.SemaphoreType.DMA,
        pltpu.SemaphoreType.DMA,
        pltpu.SemaphoreType.DMA,
    ],
)

RPW = B // NW
C = 16
NCHUNK = RPW // C
RPS = NSLOT * D // SUPW

CAT_SUP = N_CAT * C
CAT_H = CAT_SUP // 2
SEQ_SUP = SEQ_LEN * C
SEQ_H = SEQ_SUP // 2
OUT_SUP = C * RPS


def _fire_gather(tbl, idx_ref, base, n, buf, sem):
  descs = []
  off = 0
  while off < n:
    m = min(128, n - off)
    descs.append(pltpu.async_copy(
        tbl.at[idx_ref.at[pl.ds(base + off, m)]],
        buf.at[pl.ds(off, m)], sem))
    off += m
  return descs


def _body(x_hbm, w_hbm, tbl_hbm, out_hbm,
          xv, seq_v, cat_v, out_v, gidx_seq, gidx_cat, rcp_v, wv,
          sem_seq, sem_cat, sem_out):
  wid = lax.axis_index("s") * NC + lax.axis_index("c")
  lane = lax.iota(jnp.int32, 16)

  pltpu.sync_copy(w_hbm, wv)

  def chunk_body(ci, carry):
    gbase = wid * RPW + ci * C

    pltpu.sync_copy(x_hbm.at[pl.ds(gbase * NCOLS, C * NCOLS)], xv)

    rows89 = lane * NCOLS

    for j in range(N_CAT):
      v = plsc.load_gather(xv, [rows89 + (N_NUM + j)])
      gidx_cat[pl.ds(j * C, C)] = j * SPF + lax.shift_right_logical(v, 2)
    cnt = jnp.zeros((16,), jnp.float32)
    for t in range(SEQ_LEN):
      v = plsc.load_gather(xv, [rows89 + (N_NUM + N_CAT + t)])
      gidx_seq[pl.ds(t * C, C)] = (
          N_CAT * SPF + lax.shift_right_logical(v, 2))
      cnt = cnt + jnp.where(v != 0, 1.0, 0.0)
    rcp_v[pl.ds(0, 16)] = 1.0 / jnp.maximum(cnt, 1e-12)

    seq_descs = _fire_gather(tbl_hbm, gidx_seq, 0, SEQ_H, seq_v, sem_seq)
    cat_descs = _fire_gather(tbl_hbm, gidx_cat, 0, CAT_H, cat_v, sem_cat)

    for dsc in seq_descs:
      dsc.wait()

    def nrow_body(b, c2):
      xoff = b * NCOLS
      orow = b * RPS
      for i in range(N_NUM):
        xi = plsc.load_gather(
            xv, [jnp.full((16,), xoff + i, jnp.int32)]).astype(jnp.float32)
        r = orow + (i * D) // SUPW
        c0 = (i * D) % SUPW
        out_v[r, pl.ds(c0, 16)] = xi * wv[pl.ds(i * D, 16)]
        out_v[r, pl.ds(c0 + 16, 16)] = xi * wv[pl.ds(i * D + 16, 16)]
      acc0 = jnp.zeros((16,), jnp.float32)
      acc1 = jnp.zeros((16,), jnp.float32)
      for t in range(SEQ_LEN // 2):
        xq = plsc.load_gather(
            xv, [jnp.full((16,), xoff + N_NUM + N_CAT + t, jnp.int32)])
        cq = lax.bitwise_and(xq, 3) * D + lane
        rr = jnp.full((16,), t * C + b, jnp.int32)
        acc0 = acc0 + plsc.load_gather(seq_v, [rr, cq])
        acc1 = acc1 + plsc.load_gather(seq_v, [rr, cq + 16])
      out_v[orow + RPS - 1, pl.ds(96, 16)] = acc0
      out_v[orow + RPS - 1, pl.ds(112, 16)] = acc1
      return c2

    lax.fori_loop(0, C, nrow_body, 0)

    seq_descs = _fire_gather(tbl_hbm, gidx_seq, SEQ_H, SEQ_H, seq_v, sem_seq)

    for dsc in cat_descs:
      dsc.wait()

    def cat_extract(j0, b, _):
      for j in range(j0, j0 + N_CAT // 2):
        slot = N_NUM + j
        row = (j - j0) * C + b
        xq = plsc.load_gather(
            xv, [jnp.full((16,), b * NCOLS + N_NUM + j, jnp.int32)])
        cq = lax.bitwise_and(xq, 3) * D + lane
        rr = jnp.full((16,), row, jnp.int32)
        r = b * RPS + (slot * D) // SUPW
        c0 = (slot * D) % SUPW
        out_v[r, pl.ds(c0, 16)] = plsc.load_gather(cat_v, [rr, cq])
        out_v[r, pl.ds(c0 + 16, 16)] = plsc.load_gather(cat_v, [rr, cq + 16])
      return _

    lax.fori_loop(0, C, lambda b, c2: cat_extract(0, b, c2), 0)

    cat_descs = _fire_gather(tbl_hbm, gidx_cat, CAT_H, CAT_H, cat_v, sem_cat)

    for dsc in seq_descs:
      dsc.wait()

    def srow_body(b, c2):
      xoff = b * NCOLS
      acc0 = jnp.zeros((16,), jnp.float32)
      acc1 = jnp.zeros((16,), jnp.float32)
      for t in range(SEQ_LEN // 2, SEQ_LEN):
        xq = plsc.load_gather(
            xv, [jnp.full((16,), xoff + N_NUM + N_CAT + t, jnp.int32)])
        cq = lax.bitwise_and(xq, 3) * D + lane
        rr = jnp.full((16,), (t - SEQ_LEN // 2) * C + b, jnp.int32)
        acc0 = acc0 + plsc.load_gather(seq_v, [rr, cq])
        acc1 = acc1 + plsc.load_gather(seq_v, [rr, cq + 16])
      rcp = plsc.load_gather(rcp_v, [jnp.full((16,), b, jnp.int32)])
      r = b * RPS + RPS - 1
      out_v[r, pl.ds(96, 16)] = (out_v[r, pl.ds(96, 16)] + acc0) * rcp
      out_v[r, pl.ds(112, 16)] = (out_v[r, pl.ds(112, 16)] + acc1) * rcp
      return c2

    lax.fori_loop(0, C, srow_body, 0)

    for dsc in cat_descs:
      dsc.wait()
    lax.fori_loop(0, C, lambda b, c2: cat_extract(N_CAT // 2, b, c2), 0)

    pltpu.async_copy(out_v, out_hbm.at[pl.ds(gbase * RPS, OUT_SUP)],
                     sem_out).wait()
    return carry

  lax.fori_loop(0, NCHUNK, chunk_body, 0)


_sc_call = pl.kernel(
    _body,
    out_type=jax.ShapeDtypeStruct((B * RPS, SUPW), jnp.float32),
    mesh=plsc.VectorSubcoreMesh(core_axis_name="c", subcore_axis_name="s"),
    compiler_params=pltpu.CompilerParams(
        needs_layout_passes=False, use_tc_tiling_on_sc=True),
    scratch_types=[
        pltpu.VMEM((C * NCOLS,), jnp.int32),
        pltpu.VMEM((SEQ_H, SUPW), jnp.float32),
        pltpu.VMEM((CAT_H, SUPW), jnp.float32),
        pltpu.VMEM((OUT_SUP, SUPW), jnp.float32),
        pltpu.VMEM((SEQ_SUP,), jnp.int32),
        pltpu.VMEM((CAT_SUP,), jnp.int32),
        pltpu.VMEM((16,), jnp.float32),
        pltpu.VMEM((N_NUM * D,), jnp.float32),
        pltpu.SemaphoreType.DMA,
        pltpu.SemaphoreType.DMA,
        pltpu.SemaphoreType.DMA,
    ],
)


@jax.jit
def kernel(X, W_num, cat_tables, seq_table):
  cat_t = jnp.transpose(cat_tables, (0, 2, 1))
  seq_t = jnp.transpose(seq_table, (1, 0))
  nv = NFULL * VB + VT
  tail = jnp.concatenate([
      cat_tables[:, nv:, :].reshape(N_CAT * NLAST, SUPW),
      seq_table[nv:, :].reshape(NLAST, SUPW)])
  tbl = _convert(cat_t, seq_t, tail)
  out = _sc_call(X.reshape(B * NCOLS), W_num.reshape(N_NUM * D), tbl)
  return out.reshape(B, NSLOT, D)


# speedup vs baseline: 4.4341x; 1.8194x over previous
import jax
import jax.numpy as jnp
from jax import lax
from jax.experimental import pallas as pl
from jax.experimental.pallas import tpu as pltpu
from jax.experimental.pallas import tpu_sc as plsc

B = 4096
N_NUM = 13
N_CAT = 26
SEQ_LEN = 50
VOCAB = 100000
D = 32
NCOLS = N_NUM + N_CAT + SEQ_LEN
NSLOT = N_NUM + N_CAT + 1
SUPW = 128
SPF = VOCAB // 4
TBL_SUP = (N_CAT + 1) * SPF

NC, NS = 2, 16
NW = NC * NS

VB = 512
NFULL = VOCAB // VB
VT = 128
NBLK = (N_CAT + 1) * NFULL
BSUP = VB * D // SUPW
TSUP = VT * D // SUPW
NLAST = (VOCAB - NFULL * VB - VT) // 4


def _transpose_block(inb, outb, ntc, lane, prs, prsl):
  def tc_body(tc, c2):
    v0 = tc * 16
    for dt in range(2):
      rows = lane + dt * 16
      fbase = v0 * D + dt * 16
      xs = [plsc.load_gather(inb, [rows, prs[r] + v0]) for r in range(16)]
      for r in range(16):
        f = prsl[r] + fbase
        plsc.store_scatter(
            outb, [lax.shift_right_logical(f, 7), lax.bitwise_and(f, 127)],
            xs[r])
    return c2
  lax.fori_loop(0, ntc, tc_body, 0)


def _conv_body(cat_hbm, seq_hbm, tail_hbm, tbl_hbm,
               in0, in1, out0, out1, tin2, tout2, tbuf,
               sem_i0, sem_i1, sem_o0, sem_o1):
  wid = lax.axis_index("s") * NC + lax.axis_index("c")
  lane = lax.iota(jnp.int32, 16)
  prs = [lax.rem(lane + r, 16) for r in range(16)]
  prsl = [p32 * D + lane for p32 in prs]
  nblk = (NBLK - 1 - wid) // NW + 1

  ins = (in0, in1)
  outs = (out0, out1)
  sem_is = (sem_i0, sem_i1)
  sem_os = (sem_o0, sem_o1)

  def fire_in(g, buf, sem):
    bid = g * NW + wid
    f = bid // NFULL
    cb = bid - f * NFULL

    @pl.when(f < N_CAT)
    def _():
      pltpu.async_copy(cat_hbm.at[f, pl.ds(0, D), pl.ds(cb * VB, VB)],
                       buf, sem)

    @pl.when(f >= N_CAT)
    def _():
      pltpu.async_copy(seq_hbm.at[pl.ds(0, D), pl.ds(cb * VB, VB)],
                       buf, sem)

  def g_body(g, c2):
    for p in range(2):
      @pl.when(g % 2 == p)
      def _():
        @pl.when(g + 1 < nblk)
        def _():
          fire_in(g + 1, ins[1 - p], sem_is[1 - p])
        pltpu.make_async_copy(
            cat_hbm.at[0, pl.ds(0, D), pl.ds(0, VB)], ins[p],
            sem_is[p]).wait()

        @pl.when(g >= 2)
        def _():
          pltpu.make_async_copy(
              outs[p], tbl_hbm.at[pl.ds(0, BSUP)], sem_os[p]).wait()
        _transpose_block(ins[p], outs[p], VB // 16, lane, prs, prsl)
        bid = g * NW + wid
        f = bid // NFULL
        cb = bid - f * NFULL
        pltpu.async_copy(
            outs[p], tbl_hbm.at[pl.ds(f * SPF + cb * BSUP, BSUP)],
            sem_os[p])
    return c2

  fire_in(0, ins[0], sem_is[0])
  lax.fori_loop(0, nblk, g_body, 0)

  for p in range(2):
    @pl.when((nblk - 1) % 2 == p)
    def _():
      pltpu.make_async_copy(
          outs[p], tbl_hbm.at[pl.ds(0, BSUP)], sem_os[p]).wait()

    @pl.when((nblk - 2) % 2 == p)
    def _():
      pltpu.make_async_copy(
          outs[p], tbl_hbm.at[pl.ds(0, BSUP)], sem_os[p]).wait()

  @pl.when(wid <= N_CAT)
  def _():
    f = wid

    @pl.when(f < N_CAT)
    def _():
      pltpu.sync_copy(cat_hbm.at[f, pl.ds(0, D), pl.ds(NFULL * VB, VT)], tin2)

    @pl.when(f >= N_CAT)
    def _():
      pltpu.sync_copy(seq_hbm.at[pl.ds(0, D), pl.ds(NFULL * VB, VT)], tin2)
    _transpose_block(tin2, tout2, VT // 16, lane, prs, prsl)
    pltpu.sync_copy(tout2, tbl_hbm.at[pl.ds(f * SPF + NFULL * BSUP, TSUP)])
    pltpu.sync_copy(tail_hbm.at[pl.ds(f * NLAST, NLAST)], tbuf)
    pltpu.sync_copy(tbuf,
                    tbl_hbm.at[pl.ds(f * SPF + NFULL * BSUP + TSUP, NLAST)])


_convert = pl.kernel(
    _conv_body,
    out_type=jax.ShapeDtypeStruct((TBL_SUP, SUPW), jnp.float32),
    mesh=plsc.VectorSubcoreMesh(core_axis_name="c", subcore_axis_name="s"),
    compiler_params=pltpu.CompilerParams(
        needs_layout_passes=False, use_tc_tiling_on_sc=True),
    scratch_types=[
        pltpu.VMEM((D, VB), jnp.float32),
        pltpu.VMEM((D, VB), jnp.float32),
        pltpu.VMEM((BSUP, SUPW), jnp.float32),
        pltpu.VMEM((BSUP, SUPW), jnp.float32),
        pltpu.VMEM((D, VT), jnp.float32),
        pltpu.VMEM((TSUP, SUPW), jnp.float32),
        pltpu.VMEM((NLAST, SUPW), jnp.float32),
        pltpu.SemaphoreType.DMA,
        pltpu.SemaphoreType.DMA,
        pltpu.SemaphoreType.DMA,
        pltpu.SemaphoreType.DMA,
    ],
)

RPW = B // NW
C = 16
NCHUNK = RPW // C
RPS = NSLOT * D // SUPW

CAT_SUP = N_CAT * C
CAT_H = CAT_SUP // 2
SEQ_SUP = SEQ_LEN * C
SEQ_H = SEQ_SUP // 2
OUT_SUP = C * RPS


def _fire_gather(tbl, idx_ref, base, n, buf, sem):
  descs = []
  off = 0
  while off < n:
    m = min(128, n - off)
    descs.append(pltpu.async_copy(
        tbl.at[idx_ref.at[pl.ds(base + off, m)]],
        buf.at[pl.ds(off, m)], sem))
    off += m
  return descs


def _body(x_hbm, w_hbm, tbl_hbm, out_hbm,
          xv, seq_v, cat_v, out_v, gidx_seq, gidx_cat, rcp_v, wv,
          sem_seq, sem_cat, sem_out):
  wid = lax.axis_index("s") * NC + lax.axis_index("c")
  lane = lax.iota(jnp.int32, 16)

  pltpu.sync_copy(w_hbm, wv)

  def chunk_body(ci, carry):
    gbase = wid * RPW + ci * C

    pltpu.sync_copy(x_hbm.at[pl.ds(gbase * NCOLS, C * NCOLS)], xv)

    rows89 = lane * NCOLS

    for j in range(N_CAT):
      v = plsc.load_gather(xv, [rows89 + (N_NUM + j)])
      gidx_cat[pl.ds(j * C, C)] = j * SPF + lax.shift_right_logical(v, 2)
    cnt = jnp.zeros((16,), jnp.float32)
    for t in range(SEQ_LEN):
      v = plsc.load_gather(xv, [rows89 + (N_NUM + N_CAT + t)])
      gidx_seq[pl.ds(t * C, C)] = (
          N_CAT * SPF + lax.shift_right_logical(v, 2))
      cnt = cnt + jnp.where(v != 0, 1.0, 0.0)
    rcp_v[pl.ds(0, 16)] = 1.0 / jnp.maximum(cnt, 1e-12)

    seq_descs = _fire_gather(tbl_hbm, gidx_seq, 0, SEQ_H, seq_v, sem_seq)
    cat_descs = _fire_gather(tbl_hbm, gidx_cat, 0, CAT_H, cat_v, sem_cat)

    for dsc in seq_descs:
      dsc.wait()

    def nrow_body(b, c2):
      xoff = b * NCOLS
      orow = b * RPS
      for i in range(N_NUM):
        xi = plsc.load_gather(
            xv, [jnp.full((16,), xoff + i, jnp.int32)]).astype(jnp.float32)
        r = orow + (i * D) // SUPW
        c0 = (i * D) % SUPW
        out_v[r, pl.ds(c0, 16)] = xi * wv[pl.ds(i * D, 16)]
        out_v[r, pl.ds(c0 + 16, 16)] = xi * wv[pl.ds(i * D + 16, 16)]
      acc0 = jnp.zeros((16,), jnp.float32)
      acc1 = jnp.zeros((16,), jnp.float32)
      for t in range(SEQ_LEN // 2):
        xq = plsc.load_gather(
            xv, [jnp.full((16,), xoff + N_NUM + N_CAT + t, jnp.int32)])
        cq = lax.bitwise_and(xq, 3) * D + lane
        rr = jnp.full((16,), t * C + b, jnp.int32)
        acc0 = acc0 + plsc.load_gather(seq_v, [rr, cq])
        acc1 = acc1 + plsc.load_gather(seq_v, [rr, cq + 16])
      out_v[orow + RPS - 1, pl.ds(96, 16)] = acc0
      out_v[orow + RPS - 1, pl.ds(112, 16)] = acc1
      return c2

    lax.fori_loop(0, C, nrow_body, 0)

    seq_descs = _fire_gather(tbl_hbm, gidx_seq, SEQ_H, SEQ_H, seq_v, sem_seq)

    for dsc in cat_descs:
      dsc.wait()

    def cat_extract(j0, b, _):
      for j in range(j0, j0 + N_CAT // 2):
        slot = N_NUM + j
        row = (j - j0) * C + b
        xq = plsc.load_gather(
            xv, [jnp.full((16,), b * NCOLS + N_NUM + j, jnp.int32)])
        cq = lax.bitwise_and(xq, 3) * D + lane
        rr = jnp.full((16,), row, jnp.int32)
        r = b * RPS + (slot * D) // SUPW
        c0 = (slot * D) % SUPW
        out_v[r, pl.ds(c0, 16)] = plsc.load_gather(cat_v, [rr, cq])
        out_v[r, pl.ds(c0 + 16, 16)] = plsc.load_gather(cat_v, [rr, cq + 16])
      return _

    lax.fori_loop(0, C, lambda b, c2: cat_extract(0, b, c2), 0)

    cat_descs = _fire_gather(tbl_hbm, gidx_cat, CAT_H, CAT_H, cat_v, sem_cat)

    for dsc in seq_descs:
      dsc.wait()

    def srow_body(b, c2):
      xoff = b * NCOLS
      acc0 = jnp.zeros((16,), jnp.float32)
      acc1 = jnp.zeros((16,), jnp.float32)
      for t in range(SEQ_LEN // 2, SEQ_LEN):
        xq = plsc.load_gather(
            xv, [jnp.full((16,), xoff + N_NUM + N_CAT + t, jnp.int32)])
        cq = lax.bitwise_and(xq, 3) * D + lane
        rr = jnp.full((16,), (t - SEQ_LEN // 2) * C + b, jnp.int32)
        acc0 = acc0 + plsc.load_gather(seq_v, [rr, cq])
        acc1 = acc1 + plsc.load_gather(seq_v, [rr, cq + 16])
      rcp = plsc.load_gather(rcp_v, [jnp.full((16,), b, jnp.int32)])
      r = b * RPS + RPS - 1
      out_v[r, pl.ds(96, 16)] = (out_v[r, pl.ds(96, 16)] + acc0) * rcp
      out_v[r, pl.ds(112, 16)] = (out_v[r, pl.ds(112, 16)] + acc1) * rcp
      return c2

    lax.fori_loop(0, C, srow_body, 0)

    for dsc in cat_descs:
      dsc.wait()
    lax.fori_loop(0, C, lambda b, c2: cat_extract(N_CAT // 2, b, c2), 0)

    pltpu.async_copy(out_v, out_hbm.at[pl.ds(gbase * RPS, OUT_SUP)],
                     sem_out).wait()
    return carry

  lax.fori_loop(0, NCHUNK, chunk_body, 0)


_sc_call = pl.kernel(
    _body,
    out_type=jax.ShapeDtypeStruct((B * RPS, SUPW), jnp.float32),
    mesh=plsc.VectorSubcoreMesh(core_axis_name="c", subcore_axis_name="s"),
    compiler_params=pltpu.CompilerParams(
        needs_layout_passes=False, use_tc_tiling_on_sc=True),
    scratch_types=[
        pltpu.VMEM((C * NCOLS,), jnp.int32),
        pltpu.VMEM((SEQ_H, SUPW), jnp.float32),
        pltpu.VMEM((CAT_H, SUPW), jnp.float32),
        pltpu.VMEM((OUT_SUP, SUPW), jnp.float32),
        pltpu.VMEM((SEQ_SUP,), jnp.int32),
        pltpu.VMEM((CAT_SUP,), jnp.int32),
        pltpu.VMEM((16,), jnp.float32),
        pltpu.VMEM((N_NUM * D,), jnp.float32),
        pltpu.SemaphoreType.DMA,
        pltpu.SemaphoreType.DMA,
        pltpu.SemaphoreType.DMA,
    ],
)


@jax.jit
def kernel(X, W_num, cat_tables, seq_table):
  cat_t = jnp.transpose(cat_tables, (0, 2, 1))
  seq_t = jnp.transpose(seq_table, (1, 0))
  nv = NFULL * VB + VT
  tail = jnp.concatenate([
      cat_tables[:, nv:, :].reshape(N_CAT * NLAST, SUPW),
      seq_table[nv:, :].reshape(NLAST, SUPW)])
  tbl = _convert(cat_t, seq_t, tail)
  out = _sc_call(X.reshape(B * NCOLS), W_num.reshape(N_NUM * D), tbl)
  return out.reshape(B, NSLOT, D)
